# Initial kernel scaffold; baseline (speedup 1.0000x reference)
import jax
import jax.numpy as jnp
from jax.experimental import pallas as pl


def kernel(sub_logits, original_indices, full_sub_labels, full_original_indices):
    raise NotImplementedError("write your pallas kernel here")


# speedup vs baseline: 5.1657x; 5.1657x over previous
import functools

import jax
import jax.numpy as jnp
from jax import lax
from jax.experimental import pallas as pl
from jax.experimental.pallas import tpu as pltpu
from jax.experimental.pallas import tpu_sc as plsc

N = 32768
NB = 16
NS = 16
CHUNK = N // NS
ITERS = CHUNK // 16
QA = 6
ACC = QA * NB * 16


def _body(sl_hbm, oi_hbm, lab_hbm, foi_hbm, out_hbm,
          sl_v, oi_v, lab_v, foi_v, acc_v, tot_v, mrg_v, out_v, shared):
    sid = lax.axis_index("s")
    base = sid * CHUNK
    pltpu.sync_copy(sl_hbm.at[pl.ds(base * 3, CHUNK * 3)], sl_v)
    pltpu.sync_copy(oi_hbm.at[pl.ds(base, CHUNK)], oi_v)
    pltpu.sync_copy(lab_hbm.at[pl.ds(base, CHUNK)], lab_v)
    pltpu.sync_copy(foi_hbm.at[pl.ds(base, CHUNK)], foi_v)

    iota = lax.iota(jnp.int32, 16)
    zero = jnp.zeros((16,), jnp.float32)
    ones = jnp.ones((16,), jnp.float32)

    def zbody(k, _):
        acc_v[pl.ds(k * 16, 16)] = zero
        return 0
    lax.fori_loop(0, ACC // 16, zbody, 0)

    def it(i, _):
        off = i * 16
        oi = oi_v[pl.ds(off, 16)]
        b3 = i * 48 + iota * 3
        c0 = plsc.load_gather(sl_v, [b3])
        c1 = plsc.load_gather(sl_v, [b3 + 1])
        c2 = plsc.load_gather(sl_v, [b3 + 2])
        m = jnp.maximum(c0, jnp.maximum(c1, c2))
        sidx = oi * 16 + iota
        plsc.addupdate_scatter(acc_v, [sidx], ones)
        plsc.addupdate_scatter(acc_v, [sidx + 256], m)
        plsc.addupdate_scatter(acc_v, [sidx + 512], c0)
        plsc.addupdate_scatter(acc_v, [sidx + 768], c1 + c2)
        lab = lab_v[pl.ds(off, 16)]
        foi = foi_v[pl.ds(off, 16)]
        fidx = foi * 16 + iota
        plsc.addupdate_scatter(acc_v, [fidx + 1024],
                               jnp.where(lab == 4, 1.0, 0.0).astype(jnp.float32))
        plsc.addupdate_scatter(acc_v, [fidx + 1280],
                               jnp.where(lab == 1, 1.0, 0.0).astype(jnp.float32))
        return 0
    lax.fori_loop(0, ITERS, it, 0)

    for q in range(QA):
        tot = zero
        for j in range(16):
            idx = q * 256 + iota * 16 + ((iota + j) & 15)
            tot = tot + plsc.load_gather(acc_v, [idx])
        tot_v[pl.ds(q * 16, 16)] = tot

    pltpu.sync_copy(tot_v, shared.at[sid])
    plsc.subcore_barrier()

    @pl.when(sid == 0)
    def _():
        pltpu.sync_copy(shared, mrg_v)
        cnt = zero
        smax = zero
        s0 = zero
        s12 = zero
        c4 = zero
        c1n = zero
        for t in range(NS):
            row = mrg_v.at[t]
            cnt = cnt + row[pl.ds(0, 16)]
            smax = smax + row[pl.ds(16, 16)]
            s0 = s0 + row[pl.ds(32, 16)]
            s12 = s12 + row[pl.ds(48, 16)]
            c4 = c4 + row[pl.ds(64, 16)]
            c1n = c1n + row[pl.ds(80, 16)]
        avg = smax / cnt
        small = cnt < 6.0
        c4 = jnp.where(small, c4, 0.0)
        c1n = jnp.where(small, c1n, 0.0)
        x0 = s0 + c1n * avg - 5.0 * avg
        x1 = s12 + c4 * avg - avg
        j0 = 1.0 / (1.0 + jnp.exp(-x0))
        j1 = 1.0 / (1.0 + jnp.exp(-x1))
        out_v[pl.ds(0, 16)] = j0
        out_v[pl.ds(16, 16)] = j1
        pltpu.sync_copy(out_v, out_hbm)


@jax.jit
def _run(sl_flat, oi, lab, foi):
    mesh = plsc.VectorSubcoreMesh(core_axis_name="c", subcore_axis_name="s",
                                  num_cores=1)
    f = pl.kernel(
        _body,
        out_type=jax.ShapeDtypeStruct((32,), jnp.float32),
        mesh=mesh,
        compiler_params=pltpu.CompilerParams(
            use_tc_tiling_on_sc=False, needs_layout_passes=False),
        scratch_types=[
            pltpu.VMEM((CHUNK * 3,), jnp.float32),
            pltpu.VMEM((CHUNK,), jnp.int32),
            pltpu.VMEM((CHUNK,), jnp.int32),
            pltpu.VMEM((CHUNK,), jnp.int32),
            pltpu.VMEM((ACC,), jnp.float32),
            pltpu.VMEM((QA * 16,), jnp.float32),
            pltpu.VMEM((NS, QA * 16), jnp.float32),
            pltpu.VMEM((32,), jnp.float32),
            pltpu.VMEM_SHARED((NS, QA * 16), jnp.float32),
        ],
    )
    return f(sl_flat, oi, lab, foi)


def kernel(sub_logits, original_indices, full_sub_labels, full_original_indices):
    sl_flat = sub_logits.reshape(-1)
    oi = original_indices.astype(jnp.int32)
    lab = full_sub_labels.astype(jnp.int32)
    foi = full_original_indices.astype(jnp.int32)
    out = _run(sl_flat, oi, lab, foi)
    return out.reshape(2, NB).T
